# baseline (speedup 1.0000x reference)
import numpy as np
import jax
import jax.numpy as jnp
from jax import lax
from jax.experimental import pallas as pl
from jax.experimental.pallas import tpu as pltpu
from jax.experimental.pallas import tpu_sc as plsc

_VOCAB = 1000000
_EMBED = 64
_MAXLEN = 100
_BATCH = 4096
_SCALE = 8.0

_ROWS = _BATCH * _MAXLEN
_SEQ_PER_CHUNK = 4
_C = _SEQ_PER_CHUNK * _MAXLEN
_LANES = 16
_DSL = _EMBED // _LANES

_NCOL = _VOCAB // 128
_TAIL = _VOCAB - _NCOL * 128


def _pos_encoding():
    p, i = np.meshgrid(np.arange(_MAXLEN), 2 * np.arange(_EMBED // 2))
    pos = np.empty((_MAXLEN, _EMBED))
    pos[:, ::2] = np.sin(p / 10000 ** (i / _EMBED)).T
    pos[:, 1::2] = np.cos(p / 10000 ** (i / _EMBED)).T
    return pos.astype(np.float32)


def _transpose_body(nw):
    nstep = (_NCOL + nw - 1) // nw
    nhalf = (nstep + 1) // 2

    def body(tt_hbm, out_hbm, blk0, blk1, tr0, tr1, blk_t, i0, i1, o0, o1):
        cid = lax.axis_index("c")
        sid = lax.axis_index("s")
        wid = sid * 2 + cid
        blk = [blk0, blk1]
        trows = [tr0, tr1]
        isem = [i0, i1]
        osem = [o0, o1]
        iota = lax.iota(jnp.int32, _LANES)
        iota2 = iota >> 1
        parcol = (iota & 1) * 64

        def stage(i, b):
            tv = wid + nw * i

            @pl.when(tv < _NCOL)
            def _():
                pltpu.async_copy(
                    tt_hbm.at[:, pl.ds(tv * 128, 128)], blk[b], isem[b])

        def iwait(i, b):
            tv = wid + nw * i

            @pl.when(tv < _NCOL)
            def _():
                pltpu.make_async_copy(
                    tt_hbm.at[:, pl.ds(0, 128)], blk[b], isem[b]).wait()

        def owait(i, b):
            tv = wid + nw * i

            @pl.when(tv < _NCOL)
            def _():
                pltpu.make_async_copy(
                    trows[b], out_hbm.at[pl.ds(0, 64)], osem[b]).wait()

        def compute(i, b):
            tv = wid + nw * i

            @pl.when(tv < _NCOL)
            def _():
                @plsc.parallel_loop(0, _EMBED, 1, unroll=4)
                def _(e):
                    ci = parcol + e
                    for g in range(8):
                        ld = blk[b][e, pl.ds(g * _LANES, _LANES)]
                        ri = iota2 + (g * 8)
                        plsc.store_scatter(trows[b], [ri, ci], ld)

                pltpu.async_copy(
                    trows[b], out_hbm.at[pl.ds(tv * 64, 64)], osem[b])

        stage(0, 0)

        def iter_k(k, carry):
            stage(2 * k + 1, 1)
            iwait(2 * k, 0)

            @pl.when(k > 0)
            def _():
                owait(2 * k - 2, 0)

            compute(2 * k, 0)
            stage(2 * k + 2, 0)
            iwait(2 * k + 1, 1)

            @pl.when(k > 0)
            def _():
                owait(2 * k - 1, 1)

            compute(2 * k + 1, 1)
            return carry

        lax.fori_loop(0, nhalf, iter_k, 0)
        owait(nstep - 1, 0)

        @pl.when(wid == 0)
        def _():
            pltpu.sync_copy(tt_hbm.at[:, pl.ds(_NCOL * 128, _TAIL)], blk_t)

            @plsc.parallel_loop(0, _EMBED, 1, unroll=4)
            def _(e):
                ci = parcol + e
                for g in range(_TAIL // _LANES):
                    ld = blk_t[e, pl.ds(g * _LANES, _LANES)]
                    ri = iota2 + (g * 8)
                    plsc.store_scatter(tr0, [ri, ci], ld)
            pltpu.sync_copy(
                tr0.at[pl.ds(0, _TAIL // 2)],
                out_hbm.at[pl.ds(_NCOL * 64, _TAIL // 2)])

    return body


def _gather_body(nw, nchunk):
    seq_per_w = nchunk * _SEQ_PER_CHUNK

    def body(xidx_hbm, pos_hbm, table_hbm, out_hbm,
             idx0, idx1, rows0, rows1, pos_v, g0, g1, o0, o1):
        cid = lax.axis_index("c")
        sid = lax.axis_index("s")
        wid = sid * 2 + cid
        pltpu.sync_copy(pos_hbm, pos_v)

        idx = [idx0, idx1]
        rows = [rows0, rows1]
        gsem = [g0, g1]
        osem = [o0, o1]

        def stage(ci, b):
            pltpu.sync_copy(xidx_hbm.at[wid * nchunk + ci], idx[b])
            for j in range(_SEQ_PER_CHUNK):
                pltpu.async_copy(
                    table_hbm.at[idx[b].at[j]], rows[b].at[j], gsem[b])

        def gwait(b):
            for j in range(_SEQ_PER_CHUNK):
                pltpu.make_async_copy(
                    table_hbm.at[idx[b].at[j]], rows[b].at[j], gsem[b]).wait()

        def owait(b):
            pltpu.make_async_copy(
                rows[b], out_hbm.at[pl.ds(0, _SEQ_PER_CHUNK)], osem[b]).wait()

        def compute(ci, b):
            rv = rows[b]

            def lfn(l, carry):
                for d in range(_DSL):
                    sl = pl.ds(d * _LANES, _LANES)
                    p = pos_v[l, sl]
                    for s in range(_SEQ_PER_CHUNK):
                        rv[s, l, sl] = rv[s, l, sl] * _SCALE + p
                return carry

            lax.fori_loop(0, _MAXLEN, lfn, 0)
            seq0 = wid * seq_per_w + ci * _SEQ_PER_CHUNK
            pltpu.async_copy(
                rv, out_hbm.at[pl.ds(seq0, _SEQ_PER_CHUNK)], osem[b])

        nhalf = nchunk // 2
        stage(0, 0)

        def iter_k(k, carry):
            stage(2 * k + 1, 1)
            gwait(0)

            @pl.when(k > 0)
            def _():
                owait(0)

            compute(2 * k, 0)

            @pl.when(k < nhalf - 1)
            def _():
                stage(2 * k + 2, 0)

            gwait(1)

            @pl.when(k > 0)
            def _():
                owait(1)

            compute(2 * k + 1, 1)
            return carry

        lax.fori_loop(0, nhalf, iter_k, 0)
        owait(0)
        owait(1)

    return body


def kernel(x, table):
    info = plsc.get_sparse_core_info()
    nw = info.num_cores * info.num_subcores
    nchunk = _ROWS // (nw * _C)
    pos = jnp.asarray(_pos_encoding())
    x32 = x.astype(jnp.int32).reshape(nw * nchunk, _SEQ_PER_CHUNK, _MAXLEN)

    mesh = plsc.VectorSubcoreMesh(core_axis_name="c", subcore_axis_name="s")

    tfn = pl.kernel(
        _transpose_body(nw),
        mesh=mesh,
        compiler_params=pltpu.CompilerParams(
            use_tc_tiling_on_sc=True, needs_layout_passes=False),
        out_type=jax.ShapeDtypeStruct((_VOCAB // 2, 2 * _EMBED), jnp.float32),
        scratch_types=[
            pltpu.VMEM((_EMBED, 128), jnp.float32),
            pltpu.VMEM((_EMBED, 128), jnp.float32),
            pltpu.VMEM((64, 128), jnp.float32),
            pltpu.VMEM((64, 128), jnp.float32),
            pltpu.VMEM((_EMBED, _TAIL), jnp.float32),
            pltpu.SemaphoreType.DMA,
            pltpu.SemaphoreType.DMA,
            pltpu.SemaphoreType.DMA,
            pltpu.SemaphoreType.DMA,
        ],
    )
    table_rm = tfn(table.T).reshape(_VOCAB, _EMBED)

    gfn = pl.kernel(
        _gather_body(nw, nchunk),
        mesh=mesh,
        compiler_params=pltpu.CompilerParams(
            use_tc_tiling_on_sc=False, needs_layout_passes=False),
        out_type=jax.ShapeDtypeStruct((_BATCH, _MAXLEN, _EMBED), jnp.float32),
        scratch_types=[
            pltpu.VMEM((_SEQ_PER_CHUNK, _MAXLEN), jnp.int32),
            pltpu.VMEM((_SEQ_PER_CHUNK, _MAXLEN), jnp.int32),
            pltpu.VMEM((_SEQ_PER_CHUNK, _MAXLEN, _EMBED), jnp.float32),
            pltpu.VMEM((_SEQ_PER_CHUNK, _MAXLEN, _EMBED), jnp.float32),
            pltpu.VMEM((_MAXLEN, _EMBED), jnp.float32),
            pltpu.SemaphoreType.DMA,
            pltpu.SemaphoreType.DMA,
            pltpu.SemaphoreType.DMA,
            pltpu.SemaphoreType.DMA,
        ],
    )
    return gfn(x32, pos, table_rm)


# speedup vs baseline: 1.2724x; 1.2724x over previous
import numpy as np
import jax
import jax.numpy as jnp
from jax import lax
from jax.experimental import pallas as pl
from jax.experimental.pallas import tpu as pltpu
from jax.experimental.pallas import tpu_sc as plsc

_VOCAB = 1000000
_EMBED = 64
_MAXLEN = 100
_BATCH = 4096
_SCALE = 8.0

_ROWS = _BATCH * _MAXLEN
_SEQ_PER_CHUNK = 8
_C = _SEQ_PER_CHUNK * _MAXLEN
_LANES = 16
_DSL = _EMBED // _LANES


def _pos_encoding():
    p, i = np.meshgrid(np.arange(_MAXLEN), 2 * np.arange(_EMBED // 2))
    pos = np.empty((_MAXLEN, _EMBED))
    pos[:, ::2] = np.sin(p / 10000 ** (i / _EMBED)).T
    pos[:, 1::2] = np.cos(p / 10000 ** (i / _EMBED)).T
    return pos.astype(np.float32)


def _make_body(nw, nchunk):
    per_w = nchunk * _C

    def body(x_hbm, pos_hbm, table_hbm, out_hbm,
             idx0, idx1, rows0, rows1, pos_v, g0, g1, o0, o1):
        cid = lax.axis_index("c")
        sid = lax.axis_index("s")
        wid = sid * 2 + cid
        pltpu.sync_copy(pos_hbm, pos_v)

        idx = [idx0, idx1]
        rows = [rows0, rows1]
        gsem = [g0, g1]
        osem = [o0, o1]
        out_dma = [None, None]

        def stage(ci, b):
            pltpu.sync_copy(x_hbm.at[wid * nchunk + ci], idx[b])
            return [
                pltpu.async_copy(
                    table_hbm.at[idx[b].at[j]],
                    rows[b].at[pl.ds(j * _MAXLEN, _MAXLEN)],
                    gsem[b],
                )
                for j in range(_SEQ_PER_CHUNK)
            ]

        pending = [None, None]
        pending[0] = stage(0, 0)
        for ci in range(nchunk):
            b = ci & 1
            nb = b ^ 1
            if ci + 1 < nchunk:
                if out_dma[nb] is not None:
                    out_dma[nb].wait()
                    out_dma[nb] = None
                pending[nb] = stage(ci + 1, nb)
            for cpy in pending[b]:
                cpy.wait()
            rv = rows[b]

            def lfn(l, carry, rv=rv):
                for d in range(_DSL):
                    sl = pl.ds(d * _LANES, _LANES)
                    p = pos_v[l, sl]
                    for s in range(_SEQ_PER_CHUNK):
                        r = l + s * _MAXLEN
                        rv[r, sl] = rv[r, sl] * _SCALE + p
                return carry

            lax.fori_loop(0, _MAXLEN, lfn, 0)
            goff = wid * per_w + ci * _C
            out_dma[b] = pltpu.async_copy(
                rv, out_hbm.at[pl.ds(goff, _C)], osem[b])

        for b in (0, 1):
            if out_dma[b] is not None:
                out_dma[b].wait()

    return body


def kernel(x, table):
    info = plsc.get_sparse_core_info()
    nw = info.num_cores * info.num_subcores
    nchunk = _ROWS // (nw * _C)
    pos = jnp.asarray(_pos_encoding())
    x32 = x.reshape(nw * nchunk, _SEQ_PER_CHUNK, _MAXLEN).astype(jnp.int32)

    mesh = plsc.VectorSubcoreMesh(core_axis_name="c", subcore_axis_name="s")
    kfn = pl.kernel(
        _make_body(nw, nchunk),
        mesh=mesh,
        compiler_params=pltpu.CompilerParams(use_tc_tiling_on_sc=False),
        out_type=jax.ShapeDtypeStruct((_ROWS, _EMBED), jnp.float32),
        scratch_types=[
            pltpu.VMEM((_SEQ_PER_CHUNK, _MAXLEN), jnp.int32),
            pltpu.VMEM((_SEQ_PER_CHUNK, _MAXLEN), jnp.int32),
            pltpu.VMEM((_C, _EMBED), jnp.float32),
            pltpu.VMEM((_C, _EMBED), jnp.float32),
            pltpu.VMEM((_MAXLEN, _EMBED), jnp.float32),
            pltpu.SemaphoreType.DMA,
            pltpu.SemaphoreType.DMA,
            pltpu.SemaphoreType.DMA,
            pltpu.SemaphoreType.DMA,
        ],
    )
    out = kfn(x32, pos, table)
    return out.reshape(_BATCH, _MAXLEN, _EMBED)
